# Initial kernel scaffold; baseline (speedup 1.0000x reference)
import jax
import jax.numpy as jnp
from jax.experimental import pallas as pl


def kernel(x, edge_index, node_embeddings, W, att_i, att_j, gamma, beta):
    raise NotImplementedError("write your pallas kernel here")


# speedup vs baseline: 16.1869x; 16.1869x over previous
import functools

import jax
import jax.numpy as jnp
from jax import lax
from jax.experimental import pallas as pl
from jax.experimental.pallas import tpu as pltpu
from jax.experimental.pallas import tpu_sc as plsc

N = 10000
E = 320000
D = 128

NC = 2
NS = 16
NW = NC * NS
B = 128
NCHUNK = E // B
NPAD = 10240
SLC = NPAD // NS

_mesh = plsc.VectorSubcoreMesh(core_axis_name="c", subcore_axis_name="s")


_BLK = 1000
_NBLK = N // _BLK


def _prep_body(x_ref, emb_ref, w_ref, ai_ref, aj_ref, h_ref, sai_ref, saj_ref):
    h = jnp.dot(x_ref[...], w_ref[...], preferred_element_type=jnp.float32)
    h_ref[...] = h
    e = emb_ref[...]
    sai = jnp.sum(h * ai_ref[0, :][None, :], axis=1) + jnp.sum(
        e * ai_ref[1, :][None, :], axis=1)
    saj = jnp.sum(h * aj_ref[0, :][None, :], axis=1) + jnp.sum(
        e * aj_ref[1, :][None, :], axis=1)
    sai_ref[...] = sai.reshape(1, 1, _BLK)
    saj_ref[...] = saj.reshape(1, 1, _BLK)


def _prep(x, emb, w, ai2, aj2):
    return pl.pallas_call(
        _prep_body,
        grid=(_NBLK,),
        in_specs=[
            pl.BlockSpec((_BLK, D), lambda i: (i, 0)),
            pl.BlockSpec((_BLK, D), lambda i: (i, 0)),
            pl.BlockSpec((D, D), lambda i: (0, 0)),
            pl.BlockSpec((2, D), lambda i: (0, 0)),
            pl.BlockSpec((2, D), lambda i: (0, 0)),
        ],
        out_specs=[
            pl.BlockSpec((_BLK, D), lambda i: (i, 0)),
            pl.BlockSpec((1, 1, _BLK), lambda i: (i, 0, 0)),
            pl.BlockSpec((1, 1, _BLK), lambda i: (i, 0, 0)),
        ],
        out_shape=[
            jax.ShapeDtypeStruct((N, D), jnp.float32),
            jax.ShapeDtypeStruct((_NBLK, 1, _BLK), jnp.float32),
            jax.ShapeDtypeStruct((_NBLK, 1, _BLK), jnp.float32),
        ],
    )(x, emb, w, ai2, aj2)


@functools.partial(
    pl.kernel,
    out_type=[
        jax.ShapeDtypeStruct((E,), jnp.float32),
        jax.ShapeDtypeStruct((NC * NPAD,), jnp.float32),
    ],
    mesh=_mesh,
    compiler_params=pltpu.CompilerParams(needs_layout_passes=False),
    scratch_types=[
        pltpu.VMEM((NPAD,), jnp.float32),
        pltpu.VMEM((NPAD,), jnp.float32),
        pltpu.VMEM((NPAD,), jnp.float32),
        pltpu.VMEM((B,), jnp.int32),
        pltpu.VMEM((B,), jnp.int32),
        pltpu.VMEM((B,), jnp.float32),
        pltpu.VMEM((NS, SLC), jnp.float32),
        pltpu.VMEM_SHARED((NS, NPAD), jnp.float32),
    ],
)
def _edges_a(ai_hbm, aj_hbm, src_hbm, dst_hbm, ex_hbm, den_hbm,
             ai_v, aj_v, den_v, dst_v, src_v, ex_v, red_v, shared_den):
    cid = lax.axis_index("c")
    sid = lax.axis_index("s")
    wid = cid * NS + sid

    pltpu.sync_copy(ai_hbm, ai_v.at[pl.ds(0, N)])
    pltpu.sync_copy(aj_hbm, aj_v.at[pl.ds(0, N)])

    def _zero(i, _):
        den_v[pl.ds(i * 16, 16)] = jnp.zeros((16,), jnp.float32)
        return _
    lax.fori_loop(0, NPAD // 16, _zero, None)

    nch = (NCHUNK - wid + NW - 1) // NW

    def _chunk(k, _):
        base = (wid + k * NW) * B
        pltpu.sync_copy(dst_hbm.at[pl.ds(base, B)], dst_v)
        pltpu.sync_copy(src_hbm.at[pl.ds(base, B)], src_v)
        for g in range(B // 16):
            di = dst_v[pl.ds(g * 16, 16)]
            si = src_v[pl.ds(g * 16, 16)]
            al = plsc.load_gather(ai_v, [di]) + plsc.load_gather(aj_v, [si])
            al = jnp.where(al >= 0.0, al, 0.2 * al)
            exv = jnp.exp(al)
            ex_v[pl.ds(g * 16, 16)] = exv
            plsc.addupdate_scatter(den_v, [di], exv)
        pltpu.sync_copy(ex_v, ex_hbm.at[pl.ds(base, B)])
        return _
    lax.fori_loop(0, nch, _chunk, None)

    pltpu.sync_copy(den_v, shared_den.at[sid])
    plsc.subcore_barrier()
    for r in range(NS):
        pltpu.sync_copy(shared_den.at[r, pl.ds(sid * SLC, SLC)], red_v.at[r])
    for i in range(SLC // 16):
        s = red_v[0, pl.ds(i * 16, 16)]
        for r in range(1, NS):
            s = s + red_v[r, pl.ds(i * 16, 16)]
        den_v[pl.ds(i * 16, 16)] = s
    pltpu.sync_copy(den_v.at[pl.ds(0, SLC)],
                    den_hbm.at[pl.ds(cid * NPAD + sid * SLC, SLC)])


@functools.partial(
    pl.kernel,
    out_type=jax.ShapeDtypeStruct((NC * NPAD, D), jnp.float32),
    mesh=_mesh,
    compiler_params=pltpu.CompilerParams(needs_layout_passes=False),
    scratch_types=[
        pltpu.VMEM((NPAD,), jnp.float32),
        pltpu.VMEM((NPAD,), jnp.float32),
        pltpu.VMEM((B,), jnp.int32),
        pltpu.VMEM((B,), jnp.int32),
        pltpu.VMEM((B,), jnp.float32),
        pltpu.VMEM((B,), jnp.float32),
        pltpu.VMEM((B, D), jnp.float32),
        pltpu.VMEM((64, D), jnp.float32),
        pltpu.VMEM_SHARED((NPAD, D), jnp.float32),
        pltpu.SemaphoreType.DMA,
    ],
)
def _edges_b(ex_hbm, src_hbm, dst_hbm, h_hbm, den_hbm, out_hbm,
             rec_v, tmp_v, dst_v, src_v, ex_v, w_v, rows_v, stage_v,
             shared_out, sem):
    cid = lax.axis_index("c")
    sid = lax.axis_index("s")
    wid = cid * NS + sid

    pltpu.sync_copy(den_hbm.at[pl.ds(0, NPAD)], rec_v)
    pltpu.sync_copy(den_hbm.at[pl.ds(NPAD, NPAD)], tmp_v)

    def _recip(i, _):
        ds = pl.ds(i * 16, 16)
        rec_v[ds] = 1.0 / (rec_v[ds] + tmp_v[ds] + 1e-16)
        return _
    lax.fori_loop(0, NPAD // 16, _recip, None)

    def _zstage(i, _):
        r = i // (D // 16)
        j = i % (D // 16)
        stage_v[r, pl.ds(j * 16, 16)] = jnp.zeros((16,), jnp.float32)
        return _
    lax.fori_loop(0, 64 * (D // 16), _zstage, None)
    for k in range(SLC // 64):
        pltpu.sync_copy(stage_v, shared_out.at[pl.ds(sid * SLC + k * 64, 64)])
    plsc.subcore_barrier()

    nch = (NCHUNK - wid + NW - 1) // NW

    def _chunk(k, _):
        base = (wid + k * NW) * B
        pltpu.sync_copy(dst_hbm.at[pl.ds(base, B)], dst_v)
        pltpu.sync_copy(src_hbm.at[pl.ds(base, B)], src_v)
        pltpu.sync_copy(ex_hbm.at[pl.ds(base, B)], ex_v)
        pltpu.async_copy(h_hbm.at[src_v], rows_v, sem).wait()

        def _scale_grp(g, _c):
            ds16 = pl.ds(g * 16, 16)
            w_v[ds16] = ex_v[ds16] * plsc.load_gather(rec_v, [dst_v[ds16]])
            for l in range(16):
                e = g * 16 + l
                wsplat = plsc.load_gather(
                    w_v, [jnp.full((16,), e, jnp.int32)])
                for j in range(D // 16):
                    dsj = pl.ds(j * 16, 16)
                    rows_v[e, dsj] = rows_v[e, dsj] * wsplat
            return _c
        lax.fori_loop(0, B // 16, _scale_grp, None)
        pltpu.sync_copy(rows_v, shared_out.at[dst_v], add=True)
        return _
    lax.fori_loop(0, nch, _chunk, None)

    plsc.subcore_barrier()
    for k in range(SLC // 64):
        row = sid * SLC + k * 64
        pltpu.sync_copy(shared_out.at[pl.ds(row, 64)], stage_v)
        pltpu.sync_copy(stage_v, out_hbm.at[pl.ds(cid * NPAD + row, 64)])


def _bn_body(o_ref, g_ref, b_ref, out_ref):
    o = o_ref[0, :N, :] + o_ref[1, :N, :]
    mean = jnp.mean(o, axis=0)
    c = o - mean[None, :]
    var = jnp.mean(c * c, axis=0)
    y = c / jnp.sqrt(var + 1e-5)[None, :] * g_ref[0, :][None, :] \
        + b_ref[0, :][None, :]
    out_ref[...] = jnp.maximum(y, 0.0)


def _bn(out2, gamma, beta):
    return pl.pallas_call(
        _bn_body,
        out_shape=jax.ShapeDtypeStruct((N, D), jnp.float32),
    )(out2, gamma, beta)


def kernel(x, edge_index, node_embeddings, W, att_i, att_j, gamma, beta):
    src = edge_index[0]
    dst = edge_index[1]
    ai2 = att_i.reshape(2, D)
    aj2 = att_j.reshape(2, D)
    h, sai, saj = _prep(x, node_embeddings, W, ai2, aj2)
    sai = sai.reshape(N)
    saj = saj.reshape(N)
    ex, den = _edges_a(sai, saj, src, dst)
    out2 = _edges_b(ex, src, dst, h, den)
    return _bn(out2.reshape(NC, NPAD, D), gamma.reshape(1, D),
               beta.reshape(1, D))
